# baseline (speedup 1.0000x reference)
import functools

import jax
import jax.numpy as jnp
from jax import lax
from jax.experimental import pallas as pl
from jax.experimental.pallas import tpu as pltpu
from jax.experimental.pallas import tpu_sc as plsc

NUM_E = 1_000_000
D = 64
NC = 2
NS = 16
L = 16
NW = NC * NS
CHUNK = 320
NCHUNK = NUM_E // CHUNK
CW = CHUNK * D


def _scan_body(head_hbm, q_hbm, ent_hbm, vals_out, idx_out, rows_out,
               buf0, buf1, buf2, hbuf, qbuf, cv, ci, win, sem0, sem1, sem2):
    wid = lax.axis_index("s") * NC + lax.axis_index("c")
    nk = (NCHUNK - 1 - wid) // NW + 1

    pltpu.sync_copy(head_hbm, hbuf)
    pltpu.sync_copy(q_hbm, qbuf)

    h0 = hbuf[pl.ds(0, L)]
    h1 = hbuf[pl.ds(L, L)]
    h2 = hbuf[pl.ds(2 * L, L)]
    h3 = hbuf[pl.ds(3 * L, L)]
    q0 = qbuf[pl.ds(0, L)]
    q1 = qbuf[pl.ds(L, L)]
    q2 = qbuf[pl.ds(2 * L, L)]
    q3 = qbuf[pl.ds(3 * L, L)]
    wq0 = h0 * q0 - h2 * q2
    wq1 = h1 * q1 - h3 * q3
    wq2 = h2 * q0 + h0 * q2
    wq3 = h3 * q1 + h1 * q3

    def off_of(k):
        return (wid + k * NW) * CW

    def start(buf, sem, k):
        pltpu.async_copy(ent_hbm.at[pl.ds(off_of(k), CW)], buf, sem)

    def wait(buf, sem, k):
        pltpu.make_async_copy(ent_hbm.at[pl.ds(off_of(k), CW)],
                              buf, sem).wait()

    def scan_chunk(buf, r0, carry):
        @plsc.parallel_loop(0, CHUNK, step=1, unroll=8, carry=carry)
        def row_body(r, c):
            bv, bi = c
            b = r * D
            t0 = buf[pl.ds(b, L)] * wq0
            t1 = buf[pl.ds(b + L, L)] * wq1
            t2 = buf[pl.ds(b + 2 * L, L)] * wq2
            t3 = buf[pl.ds(b + 3 * L, L)] * wq3
            s = jnp.sum((t0 + t1) + (t2 + t3))
            ridx = r0 + r
            better = (s > bv) | ((s == bv) & (ridx < bi))
            bv = jnp.where(better, s, bv)
            bi = jnp.where(better, ridx, bi)
            return bv, bi
        return row_body

    bufs = (buf0, buf1, buf2)
    sems = (sem0, sem1, sem2)

    for s in range(3):
        start(bufs[s], sems[s], s)

    def outer(i3, carry):
        k0 = 3 * i3
        for s in range(3):
            buf, sem = bufs[s], sems[s]

            def do_stage(c, k=k0 + s, buf=buf, sem=sem):
                wait(buf, sem, k)
                c = scan_chunk(buf, (wid + k * NW) * CHUNK, c)

                @pl.when(k + 3 < nk)
                def _():
                    start(buf, sem, k + 3)

                return c

            carry = lax.cond(k0 + s < nk, do_stage, lambda c: c, carry)
        return carry

    n_outer = (nk + 2) // 3
    bv, bi = lax.fori_loop(0, n_outer, outer,
                           (jnp.float32(-jnp.inf), jnp.int32(0)))

    for i in range(8):
        cv[i] = jnp.full((L,), bv, jnp.float32)
        ci[i] = jnp.full((L,), bi, jnp.int32)
    pltpu.sync_copy(cv, vals_out.at[pl.ds(wid * 8, 8)])
    pltpu.sync_copy(ci, idx_out.at[pl.ds(wid * 8, 8)])
    woff = pl.multiple_of((bi // 2) * (2 * D), 2 * D)
    pltpu.sync_copy(ent_hbm.at[pl.ds(woff, 2 * D)], win)
    pltpu.sync_copy(win, rows_out.at[wid])


@functools.lru_cache(maxsize=None)
def _build_scan():
    mesh = plsc.VectorSubcoreMesh(core_axis_name="c", subcore_axis_name="s",
                                  num_cores=NC, num_subcores=NS)
    return pl.kernel(
        _scan_body,
        out_type=(
            jax.ShapeDtypeStruct((NW * 8, L), jnp.float32),
            jax.ShapeDtypeStruct((NW * 8, L), jnp.int32),
            jax.ShapeDtypeStruct((NW, 2 * D), jnp.float32),
        ),
        mesh=mesh,
        scratch_types=[
            pltpu.VMEM((CW,), jnp.float32),
            pltpu.VMEM((CW,), jnp.float32),
            pltpu.VMEM((CW,), jnp.float32),
            pltpu.VMEM((D,), jnp.float32),
            pltpu.VMEM((D,), jnp.float32),
            pltpu.VMEM((8, L), jnp.float32),
            pltpu.VMEM((8, L), jnp.int32),
            pltpu.VMEM((2 * D,), jnp.float32),
            pltpu.SemaphoreType.DMA,
            pltpu.SemaphoreType.DMA,
            pltpu.SemaphoreType.DMA,
        ],
        compiler_params=pltpu.CompilerParams(needs_layout_passes=False),
    )


def _merge_body(vals_ref, idx_ref, rows_ref, out_ref):
    vals = vals_ref[...]
    idx = idx_ref[...]
    m = jnp.max(vals)
    big = jnp.int32(jnp.iinfo(jnp.int32).max)
    hit = vals >= m
    best = jnp.min(jnp.where(hit, idx, big))
    wids = jax.lax.broadcasted_iota(jnp.int32, (NW * 8, L), 0) // 8
    wstar = jnp.min(jnp.where(hit & (idx == best), wids, big))
    rows = rows_ref[...]
    rsel = jax.lax.broadcasted_iota(jnp.int32, (NW, 2 * D), 0) == wstar
    row128 = jnp.sum(jnp.where(rsel, rows, 0.0), axis=0)
    out_ref[...] = jnp.where(best % 2 == 0, row128[:D], row128[D:])


_merge = pl.pallas_call(
    _merge_body,
    out_shape=jax.ShapeDtypeStruct((D,), jnp.float32),
    in_specs=[
        pl.BlockSpec(memory_space=pltpu.VMEM),
        pl.BlockSpec(memory_space=pltpu.VMEM),
        pl.BlockSpec(memory_space=pltpu.VMEM),
    ],
)


def kernel(head_entity, question_embedding, entity_embeddings):
    ent_flat = entity_embeddings.reshape(-1)
    vals, idx, rows = _build_scan()(head_entity, question_embedding,
                                    ent_flat)
    return _merge(vals, idx, rows)


# speedup vs baseline: 1.3430x; 1.3430x over previous
import functools

import jax
import jax.numpy as jnp
from jax import lax
from jax.experimental import pallas as pl
from jax.experimental.pallas import tpu as pltpu
from jax.experimental.pallas import tpu_sc as plsc

NUM_E = 1_000_000
D = 64
NC = 2
NS = 16
L = 16
NW = NC * NS
CHUNK = 320
NCHUNK = NUM_E // CHUNK
WIN = 16


def _scan_body(head_hbm, q_hbm, ent_hbm, vals_out, idx_out, rows_out,
               buf0, buf1, buf2, hbuf, qbuf, cv, ci, win, sem0, sem1, sem2):
    wid = lax.axis_index("s") * NC + lax.axis_index("c")
    nk = (NCHUNK - 1 - wid) // NW + 1

    pltpu.sync_copy(head_hbm, hbuf)
    pltpu.sync_copy(q_hbm, qbuf)

    h0 = hbuf[pl.ds(0, L)]
    h1 = hbuf[pl.ds(L, L)]
    h2 = hbuf[pl.ds(2 * L, L)]
    h3 = hbuf[pl.ds(3 * L, L)]
    q0 = qbuf[pl.ds(0, L)]
    q1 = qbuf[pl.ds(L, L)]
    q2 = qbuf[pl.ds(2 * L, L)]
    q3 = qbuf[pl.ds(3 * L, L)]
    wq0 = h0 * q0 - h2 * q2
    wq1 = h1 * q1 - h3 * q3
    wq2 = h2 * q0 + h0 * q2
    wq3 = h3 * q1 + h1 * q3

    lane = lax.broadcasted_iota(jnp.int32, (L,), 0)
    perms = [lane ^ k for k in (8, 4, 2, 1)]

    def hsum(v):
        for p in perms:
            v = v + v.at[p].get(mode="promise_in_bounds")
        return v[0]

    def row0_of(k):
        return (wid + k * NW) * CHUNK

    def start(buf, sem, k):
        pltpu.async_copy(ent_hbm.at[pl.ds(row0_of(k), CHUNK)], buf, sem)

    def wait(buf, sem, k):
        pltpu.make_async_copy(ent_hbm.at[pl.ds(row0_of(k), CHUNK)],
                              buf, sem).wait()

    def scan_chunk(buf, r0, carry):
        @plsc.parallel_loop(0, CHUNK, step=1, unroll=8, carry=carry)
        def row_body(r, c):
            bv, bi = c
            t0 = buf[r, pl.ds(0, L)] * wq0
            t1 = buf[r, pl.ds(L, L)] * wq1
            t2 = buf[r, pl.ds(2 * L, L)] * wq2
            t3 = buf[r, pl.ds(3 * L, L)] * wq3
            s = hsum((t0 + t1) + (t2 + t3))
            ridx = r0 + r
            better = (s > bv) | ((s == bv) & (ridx < bi))
            bv = jnp.where(better, s, bv)
            bi = jnp.where(better, ridx, bi)
            return bv, bi
        return row_body

    bufs = (buf0, buf1, buf2)
    sems = (sem0, sem1, sem2)

    for s in range(3):
        start(bufs[s], sems[s], s)

    def outer(i3, carry):
        k0 = 3 * i3
        for s in range(3):
            buf, sem = bufs[s], sems[s]

            def do_stage(c, k=k0 + s, buf=buf, sem=sem):
                wait(buf, sem, k)
                c = scan_chunk(buf, row0_of(k), c)

                @pl.when(k + 3 < nk)
                def _():
                    start(buf, sem, k + 3)

                return c

            carry = lax.cond(k0 + s < nk, do_stage, lambda c: c, carry)
        return carry

    n_outer = (nk + 2) // 3
    bv, bi = lax.fori_loop(0, n_outer, outer,
                           (jnp.float32(-jnp.inf), jnp.int32(0)))

    for i in range(8):
        cv[i] = jnp.full((L,), bv, jnp.float32)
        ci[i] = jnp.full((L,), bi, jnp.int32)
    pltpu.sync_copy(cv, vals_out.at[pl.ds(wid * 8, 8)])
    pltpu.sync_copy(ci, idx_out.at[pl.ds(wid * 8, 8)])
    wrow = pl.multiple_of((bi // WIN) * WIN, WIN)
    pltpu.sync_copy(ent_hbm.at[pl.ds(wrow, WIN)], win)
    pltpu.sync_copy(win, rows_out.at[pl.ds(wid * WIN, WIN)])


@functools.lru_cache(maxsize=None)
def _build_scan():
    mesh = plsc.VectorSubcoreMesh(core_axis_name="c", subcore_axis_name="s",
                                  num_cores=NC, num_subcores=NS)
    return pl.kernel(
        _scan_body,
        out_type=(
            jax.ShapeDtypeStruct((NW * 8, L), jnp.float32),
            jax.ShapeDtypeStruct((NW * 8, L), jnp.int32),
            jax.ShapeDtypeStruct((NW * WIN, D), jnp.float32),
        ),
        mesh=mesh,
        scratch_types=[
            pltpu.VMEM((CHUNK, D), jnp.float32),
            pltpu.VMEM((CHUNK, D), jnp.float32),
            pltpu.VMEM((CHUNK, D), jnp.float32),
            pltpu.VMEM((D,), jnp.float32),
            pltpu.VMEM((D,), jnp.float32),
            pltpu.VMEM((8, L), jnp.float32),
            pltpu.VMEM((8, L), jnp.int32),
            pltpu.VMEM((WIN, D), jnp.float32),
            pltpu.SemaphoreType.DMA,
            pltpu.SemaphoreType.DMA,
            pltpu.SemaphoreType.DMA,
        ],
    )


def _merge_body(vals_ref, idx_ref, rows_ref, out_ref):
    vals = vals_ref[...]
    idx = idx_ref[...]
    m = jnp.max(vals)
    big = jnp.int32(jnp.iinfo(jnp.int32).max)
    hit = vals >= m
    best = jnp.min(jnp.where(hit, idx, big))
    wids = jax.lax.broadcasted_iota(jnp.int32, (NW * 8, L), 0) // 8
    wstar = jnp.min(jnp.where(hit & (idx == best), wids, big))
    rows = rows_ref[...]
    gsel = wstar * WIN + best % WIN
    rsel = jax.lax.broadcasted_iota(jnp.int32, (NW * WIN, D), 0) == gsel
    out_ref[...] = jnp.sum(jnp.where(rsel, rows, 0.0), axis=0)


_merge = pl.pallas_call(
    _merge_body,
    out_shape=jax.ShapeDtypeStruct((D,), jnp.float32),
    in_specs=[
        pl.BlockSpec(memory_space=pltpu.VMEM),
        pl.BlockSpec(memory_space=pltpu.VMEM),
        pl.BlockSpec(memory_space=pltpu.VMEM),
    ],
)


def kernel(head_entity, question_embedding, entity_embeddings):
    vals, idx, rows = _build_scan()(head_entity, question_embedding,
                                    entity_embeddings)
    return _merge(vals, idx, rows)
